# Initial kernel scaffold; baseline (speedup 1.0000x reference)
import functools

import jax
import jax.numpy as jnp
from jax import lax
from jax.experimental import pallas as pl
from jax.experimental.pallas import tpu as pltpu
from jax.experimental.pallas import tpu_sc as plsc

NC = 2
NS = 16
NW = NC * NS
L = 16

N = 2_000_000
K = 1024

C0 = (N // NW) // L * L
TAIL = N - NW * C0
CT = C0 + TAIL
NV = CT // L

_mesh = plsc.VectorSubcoreMesh(core_axis_name="c", subcore_axis_name="s")


def _flags_call(labels):
    @functools.partial(
        pl.kernel,
        mesh=_mesh,
        out_type=jax.ShapeDtypeStruct((NW, K), jnp.int32),
        scratch_types=[
            pltpu.VMEM((CT,), jnp.int32),
            pltpu.VMEM((K,), jnp.int32),
        ],
    )
    def k(labels_hbm, flags_hbm, lab_v, flg_v):
        c = lax.axis_index("c")
        s = lax.axis_index("s")
        wid = s * NC + c
        base = wid * C0
        pltpu.sync_copy(labels_hbm.at[pl.ds(base, CT)], lab_v)

        zeros = jnp.zeros((L,), jnp.int32)

        def zero_body(i, carry):
            flg_v[pl.ds(i * L, L)] = zeros
            return carry

        lax.fori_loop(0, K // L, zero_body, 0)

        ones = jnp.ones((L,), jnp.int32)

        def body(i, carry):
            lv = lab_v[pl.ds(i * L, L)]
            plsc.store_scatter(flg_v, [lv - 1], ones)
            return carry

        lax.fori_loop(0, NV, body, 0)
        pltpu.sync_copy(flg_v, flags_hbm.at[wid])

    return k(labels)


def _rank_body(flags_ref, peak_ref, out_ref):
    occ = jnp.sum(flags_ref[...], axis=0)
    sub = lax.broadcasted_iota(jnp.int32, (8, 128), 0)
    lane = lax.broadcasted_iota(jnp.int32, (8, 128), 1)
    kidx = sub * 128 + lane
    occb = occ > 0
    kmax = jnp.max(jnp.where(occb, kidx + 1, 0))
    empty_in_range = jnp.logical_and(jnp.logical_not(occb), kidx < kmax)
    aa = jnp.where(empty_in_range, jnp.float32(jnp.inf), peak_ref[...])

    af = aa.reshape(K)
    ai = af[:, None]
    aj = af[None, :]
    ii = lax.broadcasted_iota(jnp.int32, (K, K), 0)
    jj = lax.broadcasted_iota(jnp.int32, (K, K), 1)
    before = jnp.logical_or(aj < ai, jnp.logical_and(aj == ai, jj < ii))
    rank = jnp.sum(before.astype(jnp.int32), axis=1)
    out_ref[...] = rank.reshape(8, 128) + 1


def _rank_call(flags, peak):
    return pl.pallas_call(
        _rank_body,
        out_shape=jax.ShapeDtypeStruct((8, 128), jnp.int32),
    )(flags.reshape(NW, 8, 128), peak.reshape(8, 128))


def _gather_call(labels, mapping):
    @functools.partial(
        pl.kernel,
        mesh=_mesh,
        out_type=jax.ShapeDtypeStruct((N,), jnp.int32),
        scratch_types=[
            pltpu.VMEM((CT,), jnp.int32),
            pltpu.VMEM((K,), jnp.int32),
        ],
    )
    def k(labels_hbm, map_hbm, out_hbm, lab_v, tab_v):
        c = lax.axis_index("c")
        s = lax.axis_index("s")
        wid = s * NC + c
        base = wid * C0
        pltpu.sync_copy(map_hbm, tab_v)
        pltpu.sync_copy(labels_hbm.at[pl.ds(base, CT)], lab_v)

        def body(i, carry):
            lv = lab_v[pl.ds(i * L, L)]
            r = plsc.load_gather(tab_v, [lv - 1])
            lab_v[pl.ds(i * L, L)] = r
            return carry

        lax.fori_loop(0, NV, body, 0)
        pltpu.sync_copy(lab_v.at[pl.ds(0, C0)], out_hbm.at[pl.ds(base, C0)])

        @pl.when(wid == NW - 1)
        def _():
            pltpu.sync_copy(
                lab_v.at[pl.ds(C0, TAIL)], out_hbm.at[pl.ds(base + C0, TAIL)]
            )

    return k(labels, mapping)


def kernel(labels, peak_channel_indices):
    flags = _flags_call(labels)
    mapping = _rank_call(flags, peak_channel_indices)
    return _gather_call(labels, mapping.reshape(K))


# speedup vs baseline: 100.7660x; 100.7660x over previous
import functools

import jax
import jax.numpy as jnp
from jax import lax
from jax.experimental import pallas as pl
from jax.experimental.pallas import tpu as pltpu
from jax.experimental.pallas import tpu_sc as plsc

NC = 2
NS = 16
NW = NC * NS
L = 16

N = 2_000_000
K = 1024

C0 = (N // NW) // L * L
TAIL = N - NW * C0
CT = C0 + TAIL
NV = CT // L

_mesh = plsc.VectorSubcoreMesh(core_axis_name="c", subcore_axis_name="s")
_sc_params = pltpu.CompilerParams(needs_layout_passes=False)


def _flags_call(labels):
    @functools.partial(
        pl.kernel,
        mesh=_mesh,
        out_type=jax.ShapeDtypeStruct((NW, K), jnp.int32),
        compiler_params=_sc_params,
        scratch_types=[
            pltpu.VMEM((CT,), jnp.int32),
            pltpu.VMEM((K,), jnp.int32),
        ],
    )
    def k(labels_hbm, flags_hbm, lab_v, flg_v):
        c = lax.axis_index("c")
        s = lax.axis_index("s")
        wid = s * NC + c
        base = wid * C0
        pltpu.sync_copy(labels_hbm.at[pl.ds(base, CT)], lab_v)

        zeros = jnp.zeros((L,), jnp.int32)

        def zero_body(i, carry):
            flg_v[pl.ds(i * L, L)] = zeros
            return carry

        lax.fori_loop(0, K // L, zero_body, 0)

        ones = jnp.ones((L,), jnp.int32)

        def body(i, carry):
            lv = lab_v[pl.ds(i * L, L)]
            plsc.store_scatter(flg_v, [lv - 1], ones)
            return carry

        lax.fori_loop(0, NV, body, 0)
        pltpu.sync_copy(flg_v, flags_hbm.at[wid])

    return k(labels)


def _aa_body(flags_ref, peak_ref, aa_ref):
    occ = jnp.sum(flags_ref[...], axis=0, keepdims=True)
    kidx = lax.broadcasted_iota(jnp.int32, (1, K), 1)
    occb = occ > 0
    kmax = jnp.max(jnp.where(occb, kidx + 1, 0))
    empty_in_range = jnp.logical_and(jnp.logical_not(occb), kidx < kmax)
    aa_ref[...] = jnp.where(empty_in_range, jnp.float32(jnp.inf), peak_ref[...])


def _rank_body(aa_row_ref, aa_col_ref, out_ref):
    aj = aa_row_ref[...]
    ai = aa_col_ref[...]
    ii = lax.broadcasted_iota(jnp.int32, (K, K), 0)
    jj = lax.broadcasted_iota(jnp.int32, (K, K), 1)
    before = jnp.logical_or(aj < ai, jnp.logical_and(aj == ai, jj < ii))
    rank = jnp.sum(before.astype(jnp.int32), axis=1, keepdims=True)
    out_ref[...] = rank + 1


def _rank_call(flags, peak):
    aa = pl.pallas_call(
        _aa_body,
        out_shape=jax.ShapeDtypeStruct((1, K), jnp.float32),
    )(flags, peak.reshape(1, K))
    mapping = pl.pallas_call(
        _rank_body,
        out_shape=jax.ShapeDtypeStruct((K, 1), jnp.int32),
    )(aa, aa.reshape(K, 1))
    return mapping


def _gather_call(labels, mapping):
    @functools.partial(
        pl.kernel,
        mesh=_mesh,
        out_type=jax.ShapeDtypeStruct((N,), jnp.int32),
        compiler_params=_sc_params,
        scratch_types=[
            pltpu.VMEM((CT,), jnp.int32),
            pltpu.VMEM((K,), jnp.int32),
        ],
    )
    def k(labels_hbm, map_hbm, out_hbm, lab_v, tab_v):
        c = lax.axis_index("c")
        s = lax.axis_index("s")
        wid = s * NC + c
        base = wid * C0
        pltpu.sync_copy(map_hbm, tab_v)
        pltpu.sync_copy(labels_hbm.at[pl.ds(base, CT)], lab_v)

        def body(i, carry):
            lv = lab_v[pl.ds(i * L, L)]
            r = plsc.load_gather(tab_v, [lv - 1])
            lab_v[pl.ds(i * L, L)] = r
            return carry

        lax.fori_loop(0, NV, body, 0)
        pltpu.sync_copy(lab_v.at[pl.ds(0, C0)], out_hbm.at[pl.ds(base, C0)])

        @pl.when(wid == NW - 1)
        def _():
            pltpu.sync_copy(
                lab_v.at[pl.ds(C0, TAIL)], out_hbm.at[pl.ds(base + C0, TAIL)]
            )

    return k(labels, mapping)


def kernel(labels, peak_channel_indices):
    flags = _flags_call(labels)
    mapping = _rank_call(flags, peak_channel_indices)
    return _gather_call(labels, mapping.reshape(K))


# speedup vs baseline: 147.7463x; 1.4662x over previous
import functools

import jax
import jax.numpy as jnp
from jax import lax
from jax.experimental import pallas as pl
from jax.experimental.pallas import tpu as pltpu
from jax.experimental.pallas import tpu_sc as plsc

NC = 2
NS = 16
NW = NC * NS
L = 16

N = 2_000_000
K = 1024

C0 = (N // NW) // L * L
TAIL = N - NW * C0
CT = C0 + TAIL

CH = 16384
_starts = list(range(0, CT, CH))
CHUNKS = [(o, min(CH, CT - o)) for o in _starts]
NCH = len(CHUNKS)
WCHUNKS = [(o, min(CH, C0 - o)) for o in _starts if o < C0]

UNROLL = 8

_mesh = plsc.VectorSubcoreMesh(core_axis_name="c", subcore_axis_name="s")
_sc_params = pltpu.CompilerParams(needs_layout_passes=False)


def _flags_call(labels):
    @functools.partial(
        pl.kernel,
        mesh=_mesh,
        out_type=jax.ShapeDtypeStruct((NW, K), jnp.int32),
        compiler_params=_sc_params,
        scratch_types=[
            pltpu.VMEM((CT,), jnp.int32),
            pltpu.VMEM((K,), jnp.int32),
        ]
        + [pltpu.SemaphoreType.DMA] * NCH,
    )
    def k(labels_hbm, flags_hbm, lab_v, flg_v, *sems):
        c = lax.axis_index("c")
        s = lax.axis_index("s")
        wid = s * NC + c
        base = wid * C0

        def in_copy(j):
            off, sz = CHUNKS[j]
            return pltpu.make_async_copy(
                labels_hbm.at[pl.ds(base + off, sz)],
                lab_v.at[pl.ds(off, sz)],
                sems[j],
            )

        for j in range(NCH):
            in_copy(j).start()

        zeros = jnp.zeros((L,), jnp.int32)
        for i in range(K // L):
            flg_v[pl.ds(i * L, L)] = zeros

        ones = jnp.ones((L,), jnp.int32)
        for j in range(NCH):
            off, sz = CHUNKS[j]
            in_copy(j).wait()

            def body(i, carry, off=off):
                lv = lab_v[pl.ds(off + i * L, L)]
                plsc.store_scatter(flg_v, [lv - 1], ones)
                return carry

            lax.fori_loop(0, sz // L, body, 0, unroll=UNROLL)

        pltpu.sync_copy(flg_v, flags_hbm.at[wid])

    return k(labels)


def _rank_body(flags_ref, peak_row_ref, peak_col_ref, out_ref):
    occ_row = (jnp.sum(flags_ref[...], axis=0, keepdims=True) > 0).astype(
        jnp.float32
    )
    kidx_row = lax.broadcasted_iota(jnp.int32, (1, K), 1)
    kmax = jnp.max(jnp.where(occ_row > 0, kidx_row + 1, 0))

    ii = lax.broadcasted_iota(jnp.int32, (K, K), 0)
    jj = lax.broadcasted_iota(jnp.int32, (K, K), 1)
    iden = (ii == jj).astype(jnp.float32)
    occ_col = lax.dot_general(
        iden,
        occ_row,
        (((1,), (1,)), ((), ())),
        preferred_element_type=jnp.float32,
    )

    inf = jnp.float32(jnp.inf)
    aa_row = jnp.where(
        jnp.logical_and(occ_row == 0.0, kidx_row < kmax), inf, peak_row_ref[...]
    )
    kidx_col = lax.broadcasted_iota(jnp.int32, (K, 1), 0)
    aa_col = jnp.where(
        jnp.logical_and(occ_col == 0.0, kidx_col < kmax), inf, peak_col_ref[...]
    )

    before = jnp.logical_or(
        aa_row < aa_col, jnp.logical_and(aa_row == aa_col, jj < ii)
    )
    rank = jnp.sum(before.astype(jnp.int32), axis=1, keepdims=True)
    out_ref[...] = rank + 1


def _rank_call(flags, peak):
    return pl.pallas_call(
        _rank_body,
        out_shape=jax.ShapeDtypeStruct((K, 1), jnp.int32),
    )(flags, peak.reshape(1, K), peak.reshape(K, 1))


def _gather_call(labels, mapping):
    @functools.partial(
        pl.kernel,
        mesh=_mesh,
        out_type=jax.ShapeDtypeStruct((N,), jnp.int32),
        compiler_params=_sc_params,
        scratch_types=[
            pltpu.VMEM((CT,), jnp.int32),
            pltpu.VMEM((K,), jnp.int32),
        ]
        + [pltpu.SemaphoreType.DMA] * (2 * NCH + 1),
    )
    def k(labels_hbm, map_hbm, out_hbm, lab_v, tab_v, *sems):
        c = lax.axis_index("c")
        s = lax.axis_index("s")
        wid = s * NC + c
        base = wid * C0
        sems_in = sems[:NCH]
        sems_out = sems[NCH : 2 * NCH]
        sem_tail = sems[2 * NCH]

        def in_copy(j):
            off, sz = CHUNKS[j]
            return pltpu.make_async_copy(
                labels_hbm.at[pl.ds(base + off, sz)],
                lab_v.at[pl.ds(off, sz)],
                sems_in[j],
            )

        def out_copy(j):
            off, sz = WCHUNKS[j]
            return pltpu.make_async_copy(
                lab_v.at[pl.ds(off, sz)],
                out_hbm.at[pl.ds(base + off, sz)],
                sems_out[j],
            )

        def tail_copy():
            return pltpu.make_async_copy(
                lab_v.at[pl.ds(C0, TAIL)],
                out_hbm.at[pl.ds(base + C0, TAIL)],
                sem_tail,
            )

        for j in range(NCH):
            in_copy(j).start()
        pltpu.sync_copy(map_hbm, tab_v)

        for j in range(NCH):
            off, sz = CHUNKS[j]
            in_copy(j).wait()

            def body(i, carry, off=off):
                lv = lab_v[pl.ds(off + i * L, L)]
                lab_v[pl.ds(off + i * L, L)] = plsc.load_gather(
                    tab_v, [lv - 1]
                )
                return carry

            lax.fori_loop(0, sz // L, body, 0, unroll=UNROLL)
            out_copy(j).start()

        @pl.when(wid == NW - 1)
        def _():
            tail_copy().start()
            tail_copy().wait()

        for j in range(NCH):
            out_copy(j).wait()

    return k(labels, mapping)


def kernel(labels, peak_channel_indices):
    flags = _flags_call(labels)
    mapping = _rank_call(flags, peak_channel_indices)
    return _gather_call(labels, mapping.reshape(K))


# speedup vs baseline: 153.1329x; 1.0365x over previous
import functools

import jax
import jax.numpy as jnp
from jax import lax
from jax.experimental import pallas as pl
from jax.experimental.pallas import tpu as pltpu
from jax.experimental.pallas import tpu_sc as plsc

NC = 2
NS = 16
NW = NC * NS
L = 16

N = 2_000_000
K = 1024

C0 = (N // NW) // L * L
TAIL = N - NW * C0
CT = C0 + TAIL

CH = 8192
_starts = list(range(0, CT, CH))
CHUNKS = [(o, min(CH, CT - o)) for o in _starts]
NCH = len(CHUNKS)
WCHUNKS = [(o, min(CH, C0 - o)) for o in _starts if o < C0]

UNROLL = 8

_mesh = plsc.VectorSubcoreMesh(core_axis_name="c", subcore_axis_name="s")
_sc_params = pltpu.CompilerParams(needs_layout_passes=False)


def _flags_call(labels):
    @functools.partial(
        pl.kernel,
        mesh=_mesh,
        out_type=jax.ShapeDtypeStruct((NW, K), jnp.int32),
        compiler_params=_sc_params,
        scratch_types=[
            pltpu.VMEM((CT,), jnp.int32),
            pltpu.VMEM((K,), jnp.int32),
        ]
        + [pltpu.SemaphoreType.DMA] * NCH,
    )
    def k(labels_hbm, flags_hbm, lab_v, flg_v, *sems):
        c = lax.axis_index("c")
        s = lax.axis_index("s")
        wid = s * NC + c
        base = wid * C0

        def in_copy(j):
            off, sz = CHUNKS[j]
            return pltpu.make_async_copy(
                labels_hbm.at[pl.ds(base + off, sz)],
                lab_v.at[pl.ds(off, sz)],
                sems[j],
            )

        for j in range(NCH):
            in_copy(j).start()

        zeros = jnp.zeros((L,), jnp.int32)
        for i in range(K // L):
            flg_v[pl.ds(i * L, L)] = zeros

        ones = jnp.ones((L,), jnp.int32)
        for j in range(NCH):
            off, sz = CHUNKS[j]
            in_copy(j).wait()

            def body(i, carry, off=off):
                lv = lab_v[pl.ds(off + i * L, L)]
                plsc.store_scatter(flg_v, [lv - 1], ones)
                return carry

            lax.fori_loop(0, sz // L, body, 0, unroll=UNROLL)

        pltpu.sync_copy(flg_v, flags_hbm.at[wid])

    return k(labels)


def _rank_body(flags_ref, peak_row_ref, peak_col_ref, out_ref):
    occ_row = (jnp.sum(flags_ref[...], axis=0, keepdims=True) > 0).astype(
        jnp.float32
    )
    kidx_row = lax.broadcasted_iota(jnp.int32, (1, K), 1)
    kmax = jnp.max(jnp.where(occ_row > 0, kidx_row + 1, 0))

    ii = lax.broadcasted_iota(jnp.int32, (K, K), 0)
    jj = lax.broadcasted_iota(jnp.int32, (K, K), 1)
    iden = (ii == jj).astype(jnp.float32)
    occ_col = lax.dot_general(
        iden,
        occ_row,
        (((1,), (1,)), ((), ())),
        preferred_element_type=jnp.float32,
    )

    inf = jnp.float32(jnp.inf)
    aa_row = jnp.where(
        jnp.logical_and(occ_row == 0.0, kidx_row < kmax), inf, peak_row_ref[...]
    )
    kidx_col = lax.broadcasted_iota(jnp.int32, (K, 1), 0)
    aa_col = jnp.where(
        jnp.logical_and(occ_col == 0.0, kidx_col < kmax), inf, peak_col_ref[...]
    )

    beforeT = jnp.logical_or(
        aa_col < aa_row, jnp.logical_and(aa_col == aa_row, ii < jj)
    )
    rank = jnp.sum(beforeT.astype(jnp.int32), axis=0, keepdims=True)
    out_ref[...] = rank + 1


def _rank_call(flags, peak):
    return pl.pallas_call(
        _rank_body,
        out_shape=jax.ShapeDtypeStruct((1, K), jnp.int32),
    )(flags, peak.reshape(1, K), peak.reshape(K, 1))


def _gather_call(labels, mapping):
    @functools.partial(
        pl.kernel,
        mesh=_mesh,
        out_type=jax.ShapeDtypeStruct((N,), jnp.int32),
        compiler_params=_sc_params,
        scratch_types=[
            pltpu.VMEM((CT,), jnp.int32),
            pltpu.VMEM((K,), jnp.int32),
        ]
        + [pltpu.SemaphoreType.DMA] * (2 * NCH + 1),
    )
    def k(labels_hbm, map_hbm, out_hbm, lab_v, tab_v, *sems):
        c = lax.axis_index("c")
        s = lax.axis_index("s")
        wid = s * NC + c
        base = wid * C0
        sems_in = sems[:NCH]
        sems_out = sems[NCH : 2 * NCH]
        sem_tail = sems[2 * NCH]

        def in_copy(j):
            off, sz = CHUNKS[j]
            return pltpu.make_async_copy(
                labels_hbm.at[pl.ds(base + off, sz)],
                lab_v.at[pl.ds(off, sz)],
                sems_in[j],
            )

        def out_copy(j):
            off, sz = WCHUNKS[j]
            return pltpu.make_async_copy(
                lab_v.at[pl.ds(off, sz)],
                out_hbm.at[pl.ds(base + off, sz)],
                sems_out[j],
            )

        def tail_copy():
            return pltpu.make_async_copy(
                lab_v.at[pl.ds(C0, TAIL)],
                out_hbm.at[pl.ds(base + C0, TAIL)],
                sem_tail,
            )

        for j in range(NCH):
            in_copy(j).start()
        pltpu.sync_copy(map_hbm, tab_v)

        for j in range(NCH):
            off, sz = CHUNKS[j]
            in_copy(j).wait()

            def body(i, carry, off=off):
                lv = lab_v[pl.ds(off + i * L, L)]
                lab_v[pl.ds(off + i * L, L)] = plsc.load_gather(
                    tab_v, [lv - 1]
                )
                return carry

            lax.fori_loop(0, sz // L, body, 0, unroll=UNROLL)
            out_copy(j).start()

        @pl.when(wid == NW - 1)
        def _():
            tail_copy().start()
            tail_copy().wait()

        for j in range(NCH):
            out_copy(j).wait()

    return k(labels, mapping)


def kernel(labels, peak_channel_indices):
    flags = _flags_call(labels)
    mapping = _rank_call(flags, peak_channel_indices)
    return _gather_call(labels, mapping.reshape(K))
